# Initial kernel scaffold; baseline (speedup 1.0000x reference)
import jax
import jax.numpy as jnp
from jax.experimental import pallas as pl


def kernel(X, node_idx, edge_idx, W1, b1, W2, b2, Wo, bo):
    raise NotImplementedError("write your pallas kernel here")


# speedup vs baseline: 2.1259x; 2.1259x over previous
import functools

import jax
import jax.numpy as jnp
from jax import lax
from jax.experimental import pallas as pl
from jax.experimental.pallas import tpu as pltpu
from jax.experimental.pallas import tpu_sc as plsc

NV = 10000
NE = 10000
NNZ = 320000
NFEAT = 128
NCLASS = 40
F3 = 64

R = 10240
DUMMY = 10000
NC, NS = 2, 16
NW = NC * NS
CH = 128
NCH = 80
NNZ_PAD = NW * NCH * CH
RPT = R // NS

_mesh = plsc.VectorSubcoreMesh(core_axis_name="c", subcore_axis_name="s",
                               num_cores=NC, num_subcores=NS)


def _make_segsum(F):
    @functools.partial(
        pl.kernel,
        out_type=jax.ShapeDtypeStruct((NC, R, F), jnp.float32),
        mesh=_mesh,
        compiler_params=pltpu.CompilerParams(use_tc_tiling_on_sc=False),
        scratch_types=[
            pltpu.VMEM((CH,), jnp.int32),
            pltpu.VMEM((CH,), jnp.int32),
            pltpu.VMEM((NCH, CH), jnp.int32),
            pltpu.VMEM((CH, F), jnp.float32),
            pltpu.VMEM((CH, F), jnp.float32),
            pltpu.VMEM_SHARED((R, F), jnp.float32),
            pltpu.SemaphoreType.DMA,
            pltpu.SemaphoreType.DMA,
            pltpu.SemaphoreType.DMA,
            pltpu.SemaphoreType.DMA,
        ],
    )
    def segsum(src, gidx, sidx, zeros, out,
               gb0, gb1, sidx_v, buf0, buf1, acc, semi0, semi1, semg0, semg1):
        c = lax.axis_index("c")
        s = lax.axis_index("s")
        wid = s * NC + c
        base = s * RPT
        pltpu.sync_copy(zeros, acc.at[pl.ds(base, RPT)])
        pltpu.sync_copy(sidx.at[wid], sidx_v)
        pltpu.make_async_copy(gidx.at[wid, 0], gb0, semi0).start()
        pltpu.make_async_copy(gidx.at[wid, 1], gb1, semi1).start()
        pltpu.make_async_copy(gidx.at[wid, 0], gb0, semi0).wait()
        pltpu.make_async_copy(src.at[gb0], buf0, semg0).start()
        pltpu.make_async_copy(gidx.at[wid, 1], gb1, semi1).wait()
        pltpu.make_async_copy(src.at[gb1], buf1, semg1).start()
        plsc.subcore_barrier()

        def body(g, carry):
            j = g * 2
            pltpu.make_async_copy(src.at[gb0], buf0, semg0).wait()
            pltpu.make_async_copy(gidx.at[wid, j + 2], gb0, semi0).start()
            pltpu.sync_copy(buf0, acc.at[sidx_v.at[j]], add=True)
            pltpu.make_async_copy(gidx.at[wid, j + 2], gb0, semi0).wait()
            pltpu.make_async_copy(src.at[gb0], buf0, semg0).start()

            pltpu.make_async_copy(src.at[gb1], buf1, semg1).wait()
            pltpu.make_async_copy(gidx.at[wid, j + 3], gb1, semi1).start()
            pltpu.sync_copy(buf1, acc.at[sidx_v.at[j + 1]], add=True)
            pltpu.make_async_copy(gidx.at[wid, j + 3], gb1, semi1).wait()
            pltpu.make_async_copy(src.at[gb1], buf1, semg1).start()
            return carry

        lax.fori_loop(0, NCH // 2, body, 0)
        pltpu.make_async_copy(src.at[gb0], buf0, semg0).wait()
        pltpu.make_async_copy(src.at[gb1], buf1, semg1).wait()
        plsc.subcore_barrier()
        pltpu.sync_copy(acc.at[pl.ds(base, RPT)], out.at[c, pl.ds(base, RPT)])

    return segsum


_segsum128 = _make_segsum(NFEAT)
_segsum64 = _make_segsum(F3)
_segsum16 = _make_segsum(16)


RB = 512
GRID = R // RB


def _recip_body(dv0, dv1, de0, de1, rv_o, re_o):
    dv = dv0[...] + dv1[...]
    de = de0[...] + de1[...]
    rv = 1.0 / jnp.maximum(dv[:, 0:1], 1.0)
    re = 1.0 / jnp.maximum(de[:, 0:1], 1.0)
    rv_o[...] = jnp.broadcast_to(rv, (RB, NFEAT))
    re_o[...] = jnp.broadcast_to(re, (RB, NFEAT))


def _recips(dv0, dv1, de0, de1):
    bs16 = pl.BlockSpec((RB, 16), lambda i: (i, 0))
    bs128 = pl.BlockSpec((RB, NFEAT), lambda i: (i, 0))
    return pl.pallas_call(
        _recip_body,
        grid=(GRID,),
        in_specs=[bs16, bs16, bs16, bs16],
        out_specs=[bs128, bs128],
        out_shape=[jax.ShapeDtypeStruct((R, NFEAT), jnp.float32)] * 2,
    )(dv0, dv1, de0, de1)


def _scale(p0, p1, r):
    F = p0.shape[1]

    def body(a, b, rr, o):
        o[...] = (a[...] + b[...]) * rr[...][:, :F]

    bsf = pl.BlockSpec((RB, F), lambda i: (i, 0))
    bs128 = pl.BlockSpec((RB, NFEAT), lambda i: (i, 0))
    return pl.pallas_call(
        body,
        grid=(GRID,),
        in_specs=[bsf, bsf, bs128],
        out_specs=bsf,
        out_shape=jax.ShapeDtypeStruct((R, F), jnp.float32),
    )(p0, p1, r)


def _mm_relu(p0, p1, rv, w, b):
    def body(a, bb, rr, w_r, b_r, o):
        hv = (a[...] + bb[...]) * rr[...]
        h = jnp.dot(hv, w_r[...], preferred_element_type=jnp.float32) + b_r[...]
        o[...] = jnp.maximum(h, 0.0)

    bs = pl.BlockSpec((RB, NFEAT), lambda i: (i, 0))
    return pl.pallas_call(
        body,
        grid=(GRID,),
        in_specs=[bs, bs, bs,
                  pl.BlockSpec((NFEAT, NFEAT), lambda i: (0, 0)),
                  pl.BlockSpec((1, NFEAT), lambda i: (0, 0))],
        out_specs=bs,
        out_shape=jax.ShapeDtypeStruct((R, NFEAT), jnp.float32),
    )(p0, p1, rv, w, b)


def _mm_relu_mm(p0, p1, rv, w, b, wo):
    def body(a, bb, rr, w_r, b_r, wo_r, o):
        hv = (a[...] + bb[...]) * rr[...]
        h = jnp.maximum(
            jnp.dot(hv, w_r[...], preferred_element_type=jnp.float32) + b_r[...],
            0.0)
        o[...] = jnp.dot(h, wo_r[...], preferred_element_type=jnp.float32)

    bs = pl.BlockSpec((RB, NFEAT), lambda i: (i, 0))
    return pl.pallas_call(
        body,
        grid=(GRID,),
        in_specs=[bs, bs, bs,
                  pl.BlockSpec((NFEAT, NFEAT), lambda i: (0, 0)),
                  pl.BlockSpec((1, NFEAT), lambda i: (0, 0)),
                  pl.BlockSpec((NFEAT, F3), lambda i: (0, 0))],
        out_specs=pl.BlockSpec((RB, F3), lambda i: (i, 0)),
        out_shape=jax.ShapeDtypeStruct((R, F3), jnp.float32),
    )(p0, p1, rv, w, b, wo)


def _final(p0, p1, rv, bo):
    def body(a, bb, rr, b_r, o):
        hv = (a[...] + bb[...]) * rr[...][:, :F3]
        logits = hv[:, :NCLASS] + b_r[...][:, :NCLASS]
        m = jnp.max(logits, axis=1, keepdims=True)
        lse = jnp.log(jnp.sum(jnp.exp(logits - m), axis=1, keepdims=True)) + m
        o[...] = logits - lse

    bsf = pl.BlockSpec((RB, F3), lambda i: (i, 0))
    return pl.pallas_call(
        body,
        grid=(GRID,),
        in_specs=[bsf, bsf,
                  pl.BlockSpec((RB, NFEAT), lambda i: (i, 0)),
                  pl.BlockSpec((1, F3), lambda i: (0, 0))],
        out_specs=pl.BlockSpec((RB, NCLASS), lambda i: (i, 0)),
        out_shape=jax.ShapeDtypeStruct((R, NCLASS), jnp.float32),
    )(p0, p1, rv, bo)


def _conv(h, ni_g, ni_s, ei_g, ei_s, zeros, rE):
    seg = _segsum128 if h.shape[1] == NFEAT else _segsum64
    p = seg(h, ni_g, ei_s, zeros)
    he = _scale(p[0], p[1], rE)
    p = seg(he, ei_g, ni_s, zeros)
    return p[0], p[1]


def kernel(X, node_idx, edge_idx, W1, b1, W2, b2, Wo, bo):
    ni = node_idx.astype(jnp.int32)
    ei = edge_idx.astype(jnp.int32)
    padv = jnp.full((NNZ_PAD - NNZ,), DUMMY, jnp.int32)
    tail = jnp.full((NW, 2, CH), DUMMY, jnp.int32)
    ni_s = jnp.concatenate([ni, padv]).reshape(NW, NCH, CH)
    ei_s = jnp.concatenate([ei, padv]).reshape(NW, NCH, CH)
    ni_g = jnp.concatenate([ni_s, tail], axis=1)
    ei_g = jnp.concatenate([ei_s, tail], axis=1)

    X_pad = jnp.zeros((R, NFEAT), jnp.float32).at[:NV].set(X)
    z128 = jnp.zeros((RPT, NFEAT), jnp.float32)
    z64 = jnp.zeros((RPT, F3), jnp.float32)
    z16 = jnp.zeros((RPT, 16), jnp.float32)
    ones_tab = jnp.ones((R, 16), jnp.float32)
    Wo_pad = jnp.zeros((NFEAT, F3), jnp.float32).at[:, :NCLASS].set(Wo)
    bo_pad = jnp.zeros((1, F3), jnp.float32).at[0, :NCLASS].set(bo)

    dV = _segsum16(ones_tab, ni_g, ni_s, z16)
    dE = _segsum16(ones_tab, ei_g, ei_s, z16)
    rV, rE = _recips(dV[0], dV[1], dE[0], dE[1])

    p0, p1 = _conv(X_pad, ni_g, ni_s, ei_g, ei_s, z128, rE)
    h = _mm_relu(p0, p1, rV, W1, b1.reshape(1, NFEAT))
    p0, p1 = _conv(h, ni_g, ni_s, ei_g, ei_s, z128, rE)
    t = _mm_relu_mm(p0, p1, rV, W2, b2.reshape(1, NFEAT), Wo_pad)
    p0, p1 = _conv(t, ni_g, ni_s, ei_g, ei_s, z64, rE)
    out = _final(p0, p1, rV, bo_pad)
    return out[:NV]
